# Initial kernel scaffold; baseline (speedup 1.0000x reference)
import jax
import jax.numpy as jnp
from jax.experimental import pallas as pl


def kernel(boxes, pred_conf):
    raise NotImplementedError("write your pallas kernel here")


# speedup vs baseline: 11.1092x; 11.1092x over previous
import functools

import jax
import jax.numpy as jnp
from jax import lax
from jax.experimental import pallas as pl
from jax.experimental.pallas import tpu as pltpu
from jax.experimental.pallas import tpu_sc as plsc

MAX_OUTPUT_SIZE = 100
IOU_THRESHOLD = 0.5
SCORE_THRESHOLD = -1e9
NEG = -1e30

N_BOXES = 20000
NPAD = 20480
NSUB = 16
CHUNK = NPAD // NSUB
NV = CHUNK // 16
NOUTV = 7
OUTPAD = 112
BIGI = 1 << 24

_mesh = plsc.VectorSubcoreMesh(
    core_axis_name="c", subcore_axis_name="s", num_cores=1, num_subcores=NSUB
)


@functools.partial(
    pl.kernel,
    out_type=(
        jax.ShapeDtypeStruct((OUTPAD,), jnp.int32),
        jax.ShapeDtypeStruct((OUTPAD,), jnp.float32),
    ),
    mesh=_mesh,
    scratch_types=[
        pltpu.VMEM((CHUNK,), jnp.float32),
        pltpu.VMEM((CHUNK + 16,), jnp.float32),
        pltpu.VMEM((CHUNK + 16,), jnp.float32),
        pltpu.VMEM((CHUNK + 16,), jnp.float32),
        pltpu.VMEM((CHUNK + 16,), jnp.float32),
        pltpu.VMEM((CHUNK,), jnp.float32),
        pltpu.VMEM((16,), jnp.float32),
        pltpu.VMEM((NSUB * 8 + 16,), jnp.float32),
        pltpu.VMEM((OUTPAD,), jnp.int32),
        pltpu.VMEM((OUTPAD,), jnp.float32),
        pltpu.VMEM_SHARED((NSUB * 8,), jnp.float32),
    ],
)
def _nms_sc(
    y1_h, x1_h, y2_h, x2_h, sc_h,
    oidx_h, osc_h,
    work, cy1, cx1, cy2, cx2, areas, pub, cand, oidx_v, osc_v, board,
):
    sid = lax.axis_index("s")
    base = sid * CHUNK
    iota = lax.iota(jnp.int32, 16)
    zeros_i = jnp.zeros((16,), jnp.int32)
    zeros_f = jnp.zeros((16,), jnp.float32)

    pltpu.sync_copy(sc_h.at[pl.ds(base, CHUNK)], work)
    pltpu.sync_copy(y1_h.at[pl.ds(base, CHUNK)], cy1.at[pl.ds(0, CHUNK)])
    pltpu.sync_copy(x1_h.at[pl.ds(base, CHUNK)], cx1.at[pl.ds(0, CHUNK)])
    pltpu.sync_copy(y2_h.at[pl.ds(base, CHUNK)], cy2.at[pl.ds(0, CHUNK)])
    pltpu.sync_copy(x2_h.at[pl.ds(base, CHUNK)], cx2.at[pl.ds(0, CHUNK)])

    for v in range(NV):
        s = pl.ds(v * 16, 16)
        areas[s] = jnp.maximum(0.0, cy2[s] - cy1[s]) * jnp.maximum(
            0.0, cx2[s] - cx1[s]
        )

    pat0 = (iota & 1) * 8
    pat1 = pat0 + 1
    pairk = iota >> 1

    def step(i, carry):
        oidx_regs, osc_regs = carry
        m = jnp.full((16,), -jnp.inf, jnp.float32)
        idxv = zeros_i + BIGI
        for v in range(NV):
            wv = work[pl.ds(v * 16, 16)]
            take = wv > m
            m = jnp.where(take, wv, m)
            idxv = jnp.where(take, base + (v * 16) + iota, idxv)
        for sh in (8, 4, 2, 1):
            perm = iota ^ sh
            m2 = m[perm]
            i2 = idxv[perm]
            better = (m2 > m) | ((m2 == m) & (i2 < idxv))
            m = jnp.where(better, m2, m)
            idxv = jnp.where(better, i2, idxv)
        li = (idxv - base)[0]
        py1 = cy1[pl.ds(li, 16)][zeros_i]
        px1 = cx1[pl.ds(li, 16)][zeros_i]
        py2 = cy2[pl.ds(li, 16)][zeros_i]
        px2 = cx2[pl.ds(li, 16)][zeros_i]
        lfv = idxv.astype(jnp.float32)
        pubv = jnp.where(
            iota == 0, m,
            jnp.where(iota == 1, lfv,
            jnp.where(iota == 2, py1,
            jnp.where(iota == 3, px1,
            jnp.where(iota == 4, py2,
            jnp.where(iota == 5, px2, 0.0))))),
        )
        pub[...] = pubv
        pltpu.sync_copy(pub.at[pl.ds(0, 8)], board.at[pl.ds(sid * 8, 8)])
        plsc.subcore_barrier()
        pltpu.sync_copy(board, cand.at[pl.ds(0, NSUB * 8)])
        plsc.subcore_barrier()

        vals = zeros_f
        idxf = zeros_f
        for k in range(8):
            rk = cand[pl.ds(k * 16, 16)]
            sel = pairk == k
            vals = jnp.where(sel, rk[pat0], vals)
            idxf = jnp.where(sel, rk[pat1], idxf)
        widv = iota
        for sh in (8, 4, 2, 1):
            perm = iota ^ sh
            v2 = vals[perm]
            f2 = idxf[perm]
            w2 = widv[perm]
            better = (v2 > vals) | ((v2 == vals) & (f2 < idxf))
            vals = jnp.where(better, v2, vals)
            idxf = jnp.where(better, f2, idxf)
            widv = jnp.where(better, w2, widv)
        best_iv = idxf.astype(jnp.int32)
        rowv = cand[pl.ds(widv[0] * 8, 16)]
        gy1 = rowv[zeros_i + 2]
        gx1 = rowv[zeros_i + 3]
        gy2 = rowv[zeros_i + 4]
        gx2 = rowv[zeros_i + 5]

        validv = vals > SCORE_THRESHOLD
        iv = zeros_i + i
        new_oidx = []
        new_osc = []
        for j in range(NOUTV):
            slot = iv == iota + (16 * j)
            new_oidx.append(
                jnp.where(slot, jnp.where(validv, best_iv, -1), oidx_regs[j])
            )
            new_osc.append(
                jnp.where(slot, jnp.where(validv, vals, 0.0), osc_regs[j])
            )

        bav = jnp.maximum(0.0, gy2 - gy1) * jnp.maximum(0.0, gx2 - gx1)
        for v in range(NV):
            s = pl.ds(v * 16, 16)
            ty1 = jnp.maximum(gy1, cy1[s])
            tx1 = jnp.maximum(gx1, cx1[s])
            ty2 = jnp.minimum(gy2, cy2[s])
            tx2 = jnp.minimum(gx2, cx2[s])
            inter = jnp.maximum(0.0, ty2 - ty1) * jnp.maximum(0.0, tx2 - tx1)
            union = bav + areas[s] - inter
            iou = jnp.where(union > 0.0, inter / union, 0.0)
            wv = work[s]
            supp = (iou > IOU_THRESHOLD) | (base + (v * 16) + iota == best_iv)
            work[s] = jnp.where(supp, NEG, wv)
        return tuple(new_oidx), tuple(new_osc)

    init = (
        tuple(zeros_i - 1 for _ in range(NOUTV)),
        tuple(zeros_f for _ in range(NOUTV)),
    )
    oidx_regs, osc_regs = lax.fori_loop(0, MAX_OUTPUT_SIZE, step, init)

    @pl.when((sid == 0) & (lax.axis_index("c") == 0))
    def _():
        for j in range(NOUTV):
            oidx_v[pl.ds(j * 16, 16)] = oidx_regs[j]
            osc_v[pl.ds(j * 16, 16)] = osc_regs[j]
        pltpu.sync_copy(oidx_v, oidx_h)
        pltpu.sync_copy(osc_v, osc_h)


def kernel(boxes, pred_conf):
    n = boxes.shape[0]
    pad = NPAD - n
    y1 = jnp.pad(boxes[:, 0], (0, pad))
    x1 = jnp.pad(boxes[:, 1], (0, pad))
    y2 = jnp.pad(boxes[:, 2], (0, pad))
    x2 = jnp.pad(boxes[:, 3], (0, pad))
    sc = jnp.pad(pred_conf.reshape(n), (0, pad), constant_values=NEG)
    oidx, osc = _nms_sc(y1, x1, y2, x2, sc)
    return oidx[:MAX_OUTPUT_SIZE], osc[:MAX_OUTPUT_SIZE]


# speedup vs baseline: 13.8256x; 1.2445x over previous
import functools

import jax
import jax.numpy as jnp
from jax import lax
from jax.experimental import pallas as pl
from jax.experimental.pallas import tpu as pltpu
from jax.experimental.pallas import tpu_sc as plsc

MAX_OUTPUT_SIZE = 100
IOU_THRESHOLD = 0.5
SCORE_THRESHOLD = -1e9
NEG = -1e30

N_BOXES = 20000
NPAD = 20480
NSUB = 16
CHUNK = NPAD // NSUB
NV = CHUNK // 16
NOUTV = 7
OUTPAD = 112
BIGI = 1 << 24

_mesh = plsc.VectorSubcoreMesh(
    core_axis_name="c", subcore_axis_name="s", num_cores=1, num_subcores=NSUB
)


@functools.partial(
    pl.kernel,
    out_type=(
        jax.ShapeDtypeStruct((OUTPAD,), jnp.int32),
        jax.ShapeDtypeStruct((OUTPAD,), jnp.float32),
    ),
    mesh=_mesh,
    scratch_types=[
        pltpu.VMEM((CHUNK,), jnp.float32),
        pltpu.VMEM((CHUNK + 16,), jnp.float32),
        pltpu.VMEM((CHUNK + 16,), jnp.float32),
        pltpu.VMEM((CHUNK + 16,), jnp.float32),
        pltpu.VMEM((CHUNK + 16,), jnp.float32),
        pltpu.VMEM((CHUNK,), jnp.float32),
        pltpu.VMEM((16,), jnp.float32),
        pltpu.VMEM((NSUB * 8 + 16,), jnp.float32),
        pltpu.VMEM((OUTPAD,), jnp.int32),
        pltpu.VMEM((OUTPAD,), jnp.float32),
        pltpu.VMEM_SHARED((NSUB * 8,), jnp.float32),
    ],
)
def _nms_sc(
    y1_h, x1_h, y2_h, x2_h, sc_h,
    oidx_h, osc_h,
    work, cy1, cx1, cy2, cx2, areas, pub, cand, oidx_v, osc_v, board,
):
    sid = lax.axis_index("s")
    base = sid * CHUNK
    iota = lax.iota(jnp.int32, 16)
    zeros_i = jnp.zeros((16,), jnp.int32)
    zeros_f = jnp.zeros((16,), jnp.float32)
    iobase = base + iota

    pltpu.sync_copy(sc_h.at[pl.ds(base, CHUNK)], work)
    pltpu.sync_copy(y1_h.at[pl.ds(base, CHUNK)], cy1.at[pl.ds(0, CHUNK)])
    pltpu.sync_copy(x1_h.at[pl.ds(base, CHUNK)], cx1.at[pl.ds(0, CHUNK)])
    pltpu.sync_copy(y2_h.at[pl.ds(base, CHUNK)], cy2.at[pl.ds(0, CHUNK)])
    pltpu.sync_copy(x2_h.at[pl.ds(base, CHUNK)], cx2.at[pl.ds(0, CHUNK)])

    for v in range(NV):
        s = pl.ds(v * 16, 16)
        areas[s] = jnp.maximum(0.0, cy2[s] - cy1[s]) * jnp.maximum(
            0.0, cx2[s] - cx1[s]
        )

    m0 = jnp.full((16,), -jnp.inf, jnp.float32)
    i0 = zeros_i + BIGI
    for v in range(NV):
        wv = work[pl.ds(v * 16, 16)]
        take = wv > m0
        m0 = jnp.where(take, wv, m0)
        i0 = jnp.where(take, iobase + (v * 16), i0)

    pat0 = (iota & 1) * 8
    pat1 = pat0 + 1
    pairk = iota >> 1
    sels = [pairk == k for k in range(8)]
    pubmasks = [iota == j for j in range(6)]

    def step(i, carry):
        m, idxv, oidx_regs, osc_regs = carry
        for sh in (8, 4, 2, 1):
            perm = iota ^ sh
            m2 = m[perm]
            i2 = idxv[perm]
            better = (m2 > m) | ((m2 == m) & (i2 < idxv))
            m = jnp.where(better, m2, m)
            idxv = jnp.where(better, i2, idxv)
        li = (idxv - base)[0]
        py1 = cy1[pl.ds(li, 16)][zeros_i]
        px1 = cx1[pl.ds(li, 16)][zeros_i]
        py2 = cy2[pl.ds(li, 16)][zeros_i]
        px2 = cx2[pl.ds(li, 16)][zeros_i]
        lfv = idxv.astype(jnp.float32)
        pubv = jnp.where(
            pubmasks[0], m,
            jnp.where(pubmasks[1], lfv,
            jnp.where(pubmasks[2], py1,
            jnp.where(pubmasks[3], px1,
            jnp.where(pubmasks[4], py2,
            jnp.where(pubmasks[5], px2, 0.0))))),
        )
        pub[...] = pubv
        pltpu.sync_copy(pub.at[pl.ds(0, 8)], board.at[pl.ds(sid * 8, 8)])
        plsc.subcore_barrier()
        pltpu.sync_copy(board, cand.at[pl.ds(0, NSUB * 8)])
        plsc.subcore_barrier()

        vals = zeros_f
        idxf = zeros_f
        for k in range(8):
            rk = cand[pl.ds(k * 16, 16)]
            vals = jnp.where(sels[k], rk[pat0], vals)
            idxf = jnp.where(sels[k], rk[pat1], idxf)
        widv = iota
        for sh in (8, 4, 2, 1):
            perm = iota ^ sh
            v2 = vals[perm]
            f2 = idxf[perm]
            w2 = widv[perm]
            better = (v2 > vals) | ((v2 == vals) & (f2 < idxf))
            vals = jnp.where(better, v2, vals)
            idxf = jnp.where(better, f2, idxf)
            widv = jnp.where(better, w2, widv)
        best_iv = idxf.astype(jnp.int32)
        rowv = cand[pl.ds(widv[0] * 8, 16)]
        gy1 = rowv[zeros_i + 2]
        gx1 = rowv[zeros_i + 3]
        gy2 = rowv[zeros_i + 4]
        gx2 = rowv[zeros_i + 5]

        validv = vals > SCORE_THRESHOLD
        iv = zeros_i + i
        new_oidx = []
        new_osc = []
        for j in range(NOUTV):
            slot = iv == iota + (16 * j)
            new_oidx.append(
                jnp.where(slot, jnp.where(validv, best_iv, -1), oidx_regs[j])
            )
            new_osc.append(
                jnp.where(slot, jnp.where(validv, vals, 0.0), osc_regs[j])
            )

        li_all = best_iv - base
        li0 = li_all[0]

        @pl.when((li0 >= 0) & (li0 < CHUNK))
        def _():
            lwa = (li_all & -16)[0]
            lmask = iota == (best_iv & 15)
            wvv = work[pl.ds(lwa, 16)]
            work[pl.ds(lwa, 16)] = jnp.where(lmask, NEG, wvv)

        bav = jnp.maximum(0.0, gy2 - gy1) * jnp.maximum(0.0, gx2 - gx1)
        nm = jnp.full((16,), -jnp.inf, jnp.float32)
        nidx = zeros_i + BIGI
        for v in range(NV):
            s = pl.ds(v * 16, 16)
            ty1 = jnp.maximum(gy1, cy1[s])
            tx1 = jnp.maximum(gx1, cx1[s])
            ty2 = jnp.minimum(gy2, cy2[s])
            tx2 = jnp.minimum(gx2, cx2[s])
            inter = jnp.maximum(0.0, ty2 - ty1) * jnp.maximum(0.0, tx2 - tx1)
            union = (bav + areas[s]) - inter
            supp = ((inter + inter) - union) * 16777216.0 > union
            wv = work[s]
            nwv = jnp.where(supp, NEG, wv)
            work[s] = nwv
            take = nwv > nm
            nm = jnp.where(take, nwv, nm)
            nidx = jnp.where(take, iobase + (v * 16), nidx)
        return nm, nidx, tuple(new_oidx), tuple(new_osc)

    init = (
        m0,
        i0,
        tuple(zeros_i - 1 for _ in range(NOUTV)),
        tuple(zeros_f for _ in range(NOUTV)),
    )
    _, _, oidx_regs, osc_regs = lax.fori_loop(0, MAX_OUTPUT_SIZE, step, init)

    @pl.when((sid == 0) & (lax.axis_index("c") == 0))
    def _():
        for j in range(NOUTV):
            oidx_v[pl.ds(j * 16, 16)] = oidx_regs[j]
            osc_v[pl.ds(j * 16, 16)] = osc_regs[j]
        pltpu.sync_copy(oidx_v, oidx_h)
        pltpu.sync_copy(osc_v, osc_h)


def kernel(boxes, pred_conf):
    n = boxes.shape[0]
    pad = NPAD - n
    y1 = jnp.pad(boxes[:, 0], (0, pad))
    x1 = jnp.pad(boxes[:, 1], (0, pad))
    y2 = jnp.pad(boxes[:, 2], (0, pad))
    x2 = jnp.pad(boxes[:, 3], (0, pad))
    sc = jnp.pad(pred_conf.reshape(n), (0, pad), constant_values=NEG)
    oidx, osc = _nms_sc(y1, x1, y2, x2, sc)
    return oidx[:MAX_OUTPUT_SIZE], osc[:MAX_OUTPUT_SIZE]


# speedup vs baseline: 14.1491x; 1.0234x over previous
import functools

import jax
import jax.numpy as jnp
from jax import lax
from jax.experimental import pallas as pl
from jax.experimental.pallas import tpu as pltpu
from jax.experimental.pallas import tpu_sc as plsc

MAX_OUTPUT_SIZE = 100
IOU_THRESHOLD = 0.5
SCORE_THRESHOLD = -1e9
NEG = -1e30

N_BOXES = 20000
NPAD = 20480
NSUB = 16
CHUNK = NPAD // NSUB
NV = CHUNK // 16
NOUTV = 7
OUTPAD = 112
BIGI = 1 << 24

_mesh = plsc.VectorSubcoreMesh(
    core_axis_name="c", subcore_axis_name="s", num_cores=1, num_subcores=NSUB
)


@functools.partial(
    pl.kernel,
    out_type=(
        jax.ShapeDtypeStruct((OUTPAD,), jnp.int32),
        jax.ShapeDtypeStruct((OUTPAD,), jnp.float32),
    ),
    mesh=_mesh,
    scratch_types=[
        pltpu.VMEM((CHUNK,), jnp.float32),
        pltpu.VMEM((CHUNK + 16,), jnp.float32),
        pltpu.VMEM((CHUNK + 16,), jnp.float32),
        pltpu.VMEM((CHUNK + 16,), jnp.float32),
        pltpu.VMEM((CHUNK + 16,), jnp.float32),
        pltpu.VMEM((CHUNK,), jnp.float32),
        pltpu.VMEM((16,), jnp.float32),
        pltpu.VMEM((NSUB * 8 + 16,), jnp.float32),
        pltpu.VMEM((OUTPAD,), jnp.int32),
        pltpu.VMEM((OUTPAD,), jnp.float32),
        pltpu.VMEM_SHARED((NSUB * 8,), jnp.float32),
    ],
)
def _nms_sc(
    y1_h, x1_h, y2_h, x2_h, sc_h,
    oidx_h, osc_h,
    work, cy1, cx1, cy2, cx2, areas, pub, cand, oidx_v, osc_v, board,
):
    sid = lax.axis_index("s")
    base = sid * CHUNK
    iota = lax.iota(jnp.int32, 16)
    zeros_i = jnp.zeros((16,), jnp.int32)
    zeros_f = jnp.zeros((16,), jnp.float32)
    iobase = base + iota

    pltpu.sync_copy(sc_h.at[pl.ds(base, CHUNK)], work)
    pltpu.sync_copy(y1_h.at[pl.ds(base, CHUNK)], cy1.at[pl.ds(0, CHUNK)])
    pltpu.sync_copy(x1_h.at[pl.ds(base, CHUNK)], cx1.at[pl.ds(0, CHUNK)])
    pltpu.sync_copy(y2_h.at[pl.ds(base, CHUNK)], cy2.at[pl.ds(0, CHUNK)])
    pltpu.sync_copy(x2_h.at[pl.ds(base, CHUNK)], cx2.at[pl.ds(0, CHUNK)])

    for v in range(NV):
        s = pl.ds(v * 16, 16)
        areas[s] = jnp.maximum(0.0, cy2[s] - cy1[s]) * jnp.maximum(
            0.0, cx2[s] - cx1[s]
        )

    m0 = jnp.full((16,), -jnp.inf, jnp.float32)
    i0 = zeros_i + BIGI
    for v in range(NV):
        wv = work[pl.ds(v * 16, 16)]
        take = wv > m0
        m0 = jnp.where(take, wv, m0)
        i0 = jnp.where(take, iobase + (v * 16), i0)

    pat0 = (iota & 1) * 8
    pat1 = pat0 + 1
    pairk = iota >> 1
    sels = [pairk == k for k in range(8)]
    pubmasks = [iota == j for j in range(6)]
    perms = [iota ^ sh for sh in (8, 4, 2, 1)]

    def step(i, carry):
        m, idxv = carry
        for perm in perms:
            m2 = m[perm]
            i2 = idxv[perm]
            better = (m2 > m) | ((m2 == m) & (i2 < idxv))
            m = jnp.where(better, m2, m)
            idxv = jnp.where(better, i2, idxv)
        li = (idxv - base)[0]
        py1 = cy1[pl.ds(li, 16)][zeros_i]
        px1 = cx1[pl.ds(li, 16)][zeros_i]
        py2 = cy2[pl.ds(li, 16)][zeros_i]
        px2 = cx2[pl.ds(li, 16)][zeros_i]
        lfv = idxv.astype(jnp.float32)
        pubv = jnp.where(
            pubmasks[0], m,
            jnp.where(pubmasks[1], lfv,
            jnp.where(pubmasks[2], py1,
            jnp.where(pubmasks[3], px1,
            jnp.where(pubmasks[4], py2,
            jnp.where(pubmasks[5], px2, 0.0))))),
        )
        pub[...] = pubv
        pltpu.sync_copy(pub.at[pl.ds(0, 8)], board.at[pl.ds(sid * 8, 8)])
        plsc.subcore_barrier()
        pltpu.sync_copy(board, cand.at[pl.ds(0, NSUB * 8)])
        plsc.subcore_barrier()

        vals = zeros_f
        idxf = zeros_f
        for k in range(8):
            rk = cand[pl.ds(k * 16, 16)]
            vals = jnp.where(sels[k], rk[pat0], vals)
            idxf = jnp.where(sels[k], rk[pat1], idxf)
        for perm in perms:
            v2 = vals[perm]
            f2 = idxf[perm]
            better = (v2 > vals) | ((v2 == vals) & (f2 < idxf))
            vals = jnp.where(better, v2, vals)
            idxf = jnp.where(better, f2, idxf)
        best_iv = idxf.astype(jnp.int32)
        wrowv = ((best_iv >> 8) * 52429) >> 18
        rowv = cand[pl.ds(wrowv[0] * 8, 16)]
        gy1 = rowv[zeros_i + 2]
        gx1 = rowv[zeros_i + 3]
        gy2 = rowv[zeros_i + 4]
        gx2 = rowv[zeros_i + 5]

        validv = vals > SCORE_THRESHOLD
        iv = zeros_i + i
        lmask_o = iota == (iv & 15)
        st = i & -16
        ov = oidx_v[pl.ds(st, 16)]
        oidx_v[pl.ds(st, 16)] = jnp.where(
            lmask_o, jnp.where(validv, best_iv, -1), ov
        )
        sv = osc_v[pl.ds(st, 16)]
        osc_v[pl.ds(st, 16)] = jnp.where(
            lmask_o, jnp.where(validv, vals, 0.0), sv
        )

        li_all = best_iv - base
        li0 = li_all[0]

        @pl.when((li0 >= 0) & (li0 < CHUNK))
        def _():
            lwa = (li_all & -16)[0]
            lmask = iota == (best_iv & 15)
            wvv = work[pl.ds(lwa, 16)]
            work[pl.ds(lwa, 16)] = jnp.where(lmask, NEG, wvv)

        bav = jnp.maximum(0.0, gy2 - gy1) * jnp.maximum(0.0, gx2 - gx1)
        nm = jnp.full((16,), -jnp.inf, jnp.float32)
        nidx = zeros_i + BIGI
        for v in range(NV):
            s = pl.ds(v * 16, 16)
            ty1 = jnp.maximum(gy1, cy1[s])
            tx1 = jnp.maximum(gx1, cx1[s])
            ty2 = jnp.minimum(gy2, cy2[s])
            tx2 = jnp.minimum(gx2, cx2[s])
            inter = jnp.maximum(0.0, ty2 - ty1) * (tx2 - tx1)
            union = (bav + areas[s]) - inter
            supp = ((inter + inter) - union) * 16777216.0 > union
            wv = work[s]
            nwv = jnp.where(supp, NEG, wv)
            work[s] = nwv
            take = nwv > nm
            nm = jnp.where(take, nwv, nm)
            nidx = jnp.where(take, iobase + (v * 16), nidx)
        return nm, nidx

    lax.fori_loop(0, MAX_OUTPUT_SIZE, step, (m0, i0))

    @pl.when((sid == 0) & (lax.axis_index("c") == 0))
    def _():
        pltpu.sync_copy(oidx_v, oidx_h)
        pltpu.sync_copy(osc_v, osc_h)


def kernel(boxes, pred_conf):
    n = boxes.shape[0]
    pad = NPAD - n
    y1 = jnp.pad(boxes[:, 0], (0, pad))
    x1 = jnp.pad(boxes[:, 1], (0, pad))
    y2 = jnp.pad(boxes[:, 2], (0, pad))
    x2 = jnp.pad(boxes[:, 3], (0, pad))
    sc = jnp.pad(pred_conf.reshape(n), (0, pad), constant_values=NEG)
    oidx, osc = _nms_sc(y1, x1, y2, x2, sc)
    return oidx[:MAX_OUTPUT_SIZE], osc[:MAX_OUTPUT_SIZE]


# speedup vs baseline: 14.7513x; 1.0426x over previous
import functools

import jax
import jax.numpy as jnp
from jax import lax
from jax.experimental import pallas as pl
from jax.experimental.pallas import tpu as pltpu
from jax.experimental.pallas import tpu_sc as plsc

MAX_OUTPUT_SIZE = 100
IOU_THRESHOLD = 0.5
SCORE_THRESHOLD = -1e9
NEG = -1e30

N_BOXES = 20000
NPAD = 20480
NSUB = 16
CHUNK = NPAD // NSUB
NV = CHUNK // 16
NOUTV = 7
OUTPAD = 112
BIGI = 1 << 24

_mesh = plsc.VectorSubcoreMesh(
    core_axis_name="c", subcore_axis_name="s", num_cores=1, num_subcores=NSUB
)


@functools.partial(
    pl.kernel,
    out_type=(
        jax.ShapeDtypeStruct((OUTPAD,), jnp.int32),
        jax.ShapeDtypeStruct((OUTPAD,), jnp.float32),
    ),
    mesh=_mesh,
    scratch_types=[
        pltpu.VMEM((CHUNK,), jnp.float32),
        pltpu.VMEM((CHUNK + 16,), jnp.float32),
        pltpu.VMEM((CHUNK + 16,), jnp.float32),
        pltpu.VMEM((CHUNK + 16,), jnp.float32),
        pltpu.VMEM((CHUNK + 16,), jnp.float32),
        pltpu.VMEM((CHUNK,), jnp.float32),
        pltpu.VMEM((16,), jnp.float32),
        pltpu.VMEM((NSUB * 8 + 16,), jnp.float32),
        pltpu.VMEM((OUTPAD,), jnp.int32),
        pltpu.VMEM((OUTPAD,), jnp.float32),
        pltpu.VMEM_SHARED((2 * NSUB * 8,), jnp.float32),
    ],
)
def _nms_sc(
    y1_h, x1_h, y2_h, x2_h, sc_h,
    oidx_h, osc_h,
    work, cy1, cx1, cy2, cx2, areas, pub, cand, oidx_v, osc_v, board,
):
    sid = lax.axis_index("s")
    base = sid * CHUNK
    iota = lax.iota(jnp.int32, 16)
    zeros_i = jnp.zeros((16,), jnp.int32)
    zeros_f = jnp.zeros((16,), jnp.float32)
    iobase = base + iota

    pltpu.sync_copy(sc_h.at[pl.ds(base, CHUNK)], work)
    pltpu.sync_copy(y1_h.at[pl.ds(base, CHUNK)], cy1.at[pl.ds(0, CHUNK)])
    pltpu.sync_copy(x1_h.at[pl.ds(base, CHUNK)], cx1.at[pl.ds(0, CHUNK)])
    pltpu.sync_copy(y2_h.at[pl.ds(base, CHUNK)], cy2.at[pl.ds(0, CHUNK)])
    pltpu.sync_copy(x2_h.at[pl.ds(base, CHUNK)], cx2.at[pl.ds(0, CHUNK)])

    for v in range(NV):
        s = pl.ds(v * 16, 16)
        areas[s] = jnp.maximum(0.0, cy2[s] - cy1[s]) * jnp.maximum(
            0.0, cx2[s] - cx1[s]
        )

    m0 = jnp.full((16,), -jnp.inf, jnp.float32)
    i0 = zeros_i + BIGI
    for v in range(NV):
        wv = work[pl.ds(v * 16, 16)]
        take = wv > m0
        m0 = jnp.where(take, wv, m0)
        i0 = jnp.where(take, iobase + (v * 16), i0)

    pat0 = (iota & 1) * 8
    pat1 = pat0 + 1
    pairk = iota >> 1
    sels = [pairk == k for k in range(8)]
    pubmasks = [iota == j for j in range(6)]
    perms = [iota ^ sh for sh in (8, 4, 2, 1)]

    def step(i, carry):
        m, idxv = carry
        for perm in perms:
            m2 = m[perm]
            i2 = idxv[perm]
            better = (m2 > m) | ((m2 == m) & (i2 < idxv))
            m = jnp.where(better, m2, m)
            idxv = jnp.where(better, i2, idxv)
        li = (idxv - base)[0]
        py1 = cy1[pl.ds(li, 16)][zeros_i]
        px1 = cx1[pl.ds(li, 16)][zeros_i]
        py2 = cy2[pl.ds(li, 16)][zeros_i]
        px2 = cx2[pl.ds(li, 16)][zeros_i]
        lfv = idxv.astype(jnp.float32)
        pubv = jnp.where(
            pubmasks[0], m,
            jnp.where(pubmasks[1], lfv,
            jnp.where(pubmasks[2], py1,
            jnp.where(pubmasks[3], px1,
            jnp.where(pubmasks[4], py2,
            jnp.where(pubmasks[5], px2, 0.0))))),
        )
        pub[...] = pubv
        boff = (i & 1) * (NSUB * 8)
        pltpu.sync_copy(pub.at[pl.ds(0, 8)], board.at[pl.ds(boff + sid * 8, 8)])
        plsc.subcore_barrier()
        pltpu.sync_copy(
            board.at[pl.ds(boff, NSUB * 8)], cand.at[pl.ds(0, NSUB * 8)]
        )

        vals = zeros_f
        idxf = zeros_f
        for k in range(8):
            rk = cand[pl.ds(k * 16, 16)]
            vals = jnp.where(sels[k], rk[pat0], vals)
            idxf = jnp.where(sels[k], rk[pat1], idxf)
        for perm in perms:
            v2 = vals[perm]
            f2 = idxf[perm]
            better = (v2 > vals) | ((v2 == vals) & (f2 < idxf))
            vals = jnp.where(better, v2, vals)
            idxf = jnp.where(better, f2, idxf)
        best_iv = idxf.astype(jnp.int32)
        wrowv = ((best_iv >> 8) * 52429) >> 18
        rowv = cand[pl.ds(wrowv[0] * 8, 16)]
        gy1 = rowv[zeros_i + 2]
        gx1 = rowv[zeros_i + 3]
        gy2 = rowv[zeros_i + 4]
        gx2 = rowv[zeros_i + 5]

        validv = vals > SCORE_THRESHOLD
        iv = zeros_i + i
        lmask_o = iota == (iv & 15)
        st = i & -16
        ov = oidx_v[pl.ds(st, 16)]
        oidx_v[pl.ds(st, 16)] = jnp.where(
            lmask_o, jnp.where(validv, best_iv, -1), ov
        )
        sv = osc_v[pl.ds(st, 16)]
        osc_v[pl.ds(st, 16)] = jnp.where(
            lmask_o, jnp.where(validv, vals, 0.0), sv
        )

        li_all = best_iv - base
        li0 = li_all[0]

        @pl.when((li0 >= 0) & (li0 < CHUNK))
        def _():
            lwa = (li_all & -16)[0]
            lmask = iota == (best_iv & 15)
            wvv = work[pl.ds(lwa, 16)]
            work[pl.ds(lwa, 16)] = jnp.where(lmask, NEG, wvv)

        bav = jnp.maximum(0.0, gy2 - gy1) * jnp.maximum(0.0, gx2 - gx1)
        nm = jnp.full((16,), -jnp.inf, jnp.float32)
        nidx = zeros_i + BIGI
        for v in range(NV):
            s = pl.ds(v * 16, 16)
            ty1 = jnp.maximum(gy1, cy1[s])
            tx1 = jnp.maximum(gx1, cx1[s])
            ty2 = jnp.minimum(gy2, cy2[s])
            tx2 = jnp.minimum(gx2, cx2[s])
            inter = jnp.maximum(0.0, ty2 - ty1) * (tx2 - tx1)
            union = (bav + areas[s]) - inter
            supp = ((inter + inter) - union) * 16777216.0 > union
            wv = work[s]
            nwv = jnp.where(supp, NEG, wv)
            work[s] = nwv
            take = nwv > nm
            nm = jnp.where(take, nwv, nm)
            nidx = jnp.where(take, iobase + (v * 16), nidx)
        return nm, nidx

    lax.fori_loop(0, MAX_OUTPUT_SIZE, step, (m0, i0))

    @pl.when((sid == 0) & (lax.axis_index("c") == 0))
    def _():
        pltpu.sync_copy(oidx_v, oidx_h)
        pltpu.sync_copy(osc_v, osc_h)


def kernel(boxes, pred_conf):
    n = boxes.shape[0]
    pad = NPAD - n
    y1 = jnp.pad(boxes[:, 0], (0, pad))
    x1 = jnp.pad(boxes[:, 1], (0, pad))
    y2 = jnp.pad(boxes[:, 2], (0, pad))
    x2 = jnp.pad(boxes[:, 3], (0, pad))
    sc = jnp.pad(pred_conf.reshape(n), (0, pad), constant_values=NEG)
    oidx, osc = _nms_sc(y1, x1, y2, x2, sc)
    return oidx[:MAX_OUTPUT_SIZE], osc[:MAX_OUTPUT_SIZE]


# speedup vs baseline: 15.3973x; 1.0438x over previous
import functools

import jax
import jax.numpy as jnp
from jax import lax
from jax.experimental import pallas as pl
from jax.experimental.pallas import tpu as pltpu
from jax.experimental.pallas import tpu_sc as plsc

MAX_OUTPUT_SIZE = 100
IOU_THRESHOLD = 0.5
SCORE_THRESHOLD = -1e9
NEG = -1e30

N_BOXES = 20000
NPAD = 20480
NSUB = 16
CHUNK = NPAD // NSUB
NV = CHUNK // 16
NOUTV = 7
OUTPAD = 112
BIGI = 1 << 24

_mesh = plsc.VectorSubcoreMesh(
    core_axis_name="c", subcore_axis_name="s", num_cores=1, num_subcores=NSUB
)


@functools.partial(
    pl.kernel,
    out_type=(
        jax.ShapeDtypeStruct((OUTPAD,), jnp.int32),
        jax.ShapeDtypeStruct((OUTPAD,), jnp.float32),
    ),
    mesh=_mesh,
    scratch_types=[
        pltpu.VMEM((CHUNK,), jnp.float32),
        pltpu.VMEM((CHUNK + 16,), jnp.float32),
        pltpu.VMEM((CHUNK + 16,), jnp.float32),
        pltpu.VMEM((CHUNK + 16,), jnp.float32),
        pltpu.VMEM((CHUNK + 16,), jnp.float32),
        pltpu.VMEM((CHUNK,), jnp.float32),
        pltpu.VMEM((16,), jnp.float32),
        pltpu.VMEM((NSUB * 8 + 16,), jnp.float32),
        pltpu.VMEM((OUTPAD,), jnp.int32),
        pltpu.VMEM((OUTPAD,), jnp.float32),
        pltpu.VMEM_SHARED((2 * NSUB * 8,), jnp.float32),
    ],
)
def _nms_sc(
    y1_h, x1_h, y2_h, x2_h, sc_h,
    oidx_h, osc_h,
    work, cy1, cx1, cy2, cx2, areas, pub, cand, oidx_v, osc_v, board,
):
    sid = lax.axis_index("s")
    base = sid * CHUNK
    iota = lax.iota(jnp.int32, 16)
    zeros_i = jnp.zeros((16,), jnp.int32)
    zeros_f = jnp.zeros((16,), jnp.float32)
    iobase = base + iota

    pltpu.sync_copy(sc_h.at[pl.ds(base, CHUNK)], work)
    pltpu.sync_copy(y1_h.at[pl.ds(base, CHUNK)], cy1.at[pl.ds(0, CHUNK)])
    pltpu.sync_copy(x1_h.at[pl.ds(base, CHUNK)], cx1.at[pl.ds(0, CHUNK)])
    pltpu.sync_copy(y2_h.at[pl.ds(base, CHUNK)], cy2.at[pl.ds(0, CHUNK)])
    pltpu.sync_copy(x2_h.at[pl.ds(base, CHUNK)], cx2.at[pl.ds(0, CHUNK)])

    for v in range(NV):
        s = pl.ds(v * 16, 16)
        areas[s] = jnp.maximum(0.0, cy2[s] - cy1[s]) * jnp.maximum(
            0.0, cx2[s] - cx1[s]
        )

    m0 = jnp.full((16,), -jnp.inf, jnp.float32)
    i0 = zeros_i + BIGI
    for v in range(NV):
        wv = work[pl.ds(v * 16, 16)]
        take = wv > m0
        m0 = jnp.where(take, wv, m0)
        i0 = jnp.where(take, iobase + (v * 16), i0)

    pat0 = (iota & 1) * 8
    pat1 = pat0 + 1
    pairk = iota >> 1
    sels = [pairk == k for k in range(8)]
    pubmasks = [iota == j for j in range(6)]
    perms = [iota ^ sh for sh in (8, 4, 2, 1)]

    def step(i, carry):
        m, idxv = carry
        for perm in perms:
            m2 = m[perm]
            i2 = idxv[perm]
            better = (m2 > m) | ((m2 == m) & (i2 < idxv))
            m = jnp.where(better, m2, m)
            idxv = jnp.where(better, i2, idxv)
        li = (idxv - base)[0]
        py1 = cy1[pl.ds(li, 16)][zeros_i]
        px1 = cx1[pl.ds(li, 16)][zeros_i]
        py2 = cy2[pl.ds(li, 16)][zeros_i]
        px2 = cx2[pl.ds(li, 16)][zeros_i]
        lfv = idxv.astype(jnp.float32)
        pubv = jnp.where(
            pubmasks[0], m,
            jnp.where(pubmasks[1], lfv,
            jnp.where(pubmasks[2], py1,
            jnp.where(pubmasks[3], px1,
            jnp.where(pubmasks[4], py2,
            jnp.where(pubmasks[5], px2, 0.0))))),
        )
        pub[...] = pubv
        boff = (i & 1) * (NSUB * 8)
        pltpu.sync_copy(pub.at[pl.ds(0, 8)], board.at[pl.ds(boff + sid * 8, 8)])
        plsc.subcore_barrier()
        pltpu.sync_copy(
            board.at[pl.ds(boff, NSUB * 8)], cand.at[pl.ds(0, NSUB * 8)]
        )

        vals = zeros_f
        idxf = zeros_f
        for k in range(8):
            rk = cand[pl.ds(k * 16, 16)]
            vals = jnp.where(sels[k], rk[pat0], vals)
            idxf = jnp.where(sels[k], rk[pat1], idxf)
        for perm in perms:
            v2 = vals[perm]
            f2 = idxf[perm]
            better = (v2 > vals) | ((v2 == vals) & (f2 < idxf))
            vals = jnp.where(better, v2, vals)
            idxf = jnp.where(better, f2, idxf)
        best_iv = idxf.astype(jnp.int32)
        wrowv = ((best_iv >> 8) * 52429) >> 18
        rowv = cand[pl.ds(wrowv[0] * 8, 16)]
        gy1 = rowv[zeros_i + 2]
        gx1 = rowv[zeros_i + 3]
        gy2 = rowv[zeros_i + 4]
        gx2 = rowv[zeros_i + 5]

        validv = vals > SCORE_THRESHOLD
        iv = zeros_i + i
        lmask_o = iota == (iv & 15)
        st = i & -16
        ov = oidx_v[pl.ds(st, 16)]
        oidx_v[pl.ds(st, 16)] = jnp.where(
            lmask_o, jnp.where(validv, best_iv, -1), ov
        )
        sv = osc_v[pl.ds(st, 16)]
        osc_v[pl.ds(st, 16)] = jnp.where(
            lmask_o, jnp.where(validv, vals, 0.0), sv
        )

        li_all = best_iv - base
        li0 = li_all[0]

        @pl.when((li0 >= 0) & (li0 < CHUNK))
        def _():
            lwa = (li_all & -16)[0]
            lmask = iota == (best_iv & 15)
            wvv = work[pl.ds(lwa, 16)]
            work[pl.ds(lwa, 16)] = jnp.where(lmask, NEG, wvv)

        bav = jnp.maximum(0.0, gy2 - gy1) * jnp.maximum(0.0, gx2 - gx1)
        nm = jnp.full((16,), -jnp.inf, jnp.float32)
        nidx = zeros_i + BIGI
        for v in range(NV):
            s = pl.ds(v * 16, 16)
            ty1 = jnp.maximum(gy1, cy1[s])
            tx1 = jnp.maximum(gx1, cx1[s])
            ty2 = jnp.minimum(gy2, cy2[s])
            tx2 = jnp.minimum(gx2, cx2[s])
            inter = jnp.maximum(0.0, ty2 - ty1) * (tx2 - tx1)
            union = (bav + areas[s]) - inter
            supp = ((inter + inter) - union) * 16777216.0 > union
            wv = work[s]
            nwv = jnp.where(supp, NEG, wv)
            work[s] = nwv
            take = nwv > nm
            nm = jnp.where(take, nwv, nm)
            nidx = jnp.where(take, v, nidx)
        return nm, (nidx << 4) + iobase

    lax.fori_loop(0, MAX_OUTPUT_SIZE, step, (m0, i0))

    @pl.when((sid == 0) & (lax.axis_index("c") == 0))
    def _():
        pltpu.sync_copy(oidx_v, oidx_h)
        pltpu.sync_copy(osc_v, osc_h)


def kernel(boxes, pred_conf):
    n = boxes.shape[0]
    pad = NPAD - n
    y1 = jnp.pad(boxes[:, 0], (0, pad))
    x1 = jnp.pad(boxes[:, 1], (0, pad))
    y2 = jnp.pad(boxes[:, 2], (0, pad))
    x2 = jnp.pad(boxes[:, 3], (0, pad))
    sc = jnp.pad(pred_conf.reshape(n), (0, pad), constant_values=NEG)
    oidx, osc = _nms_sc(y1, x1, y2, x2, sc)
    return oidx[:MAX_OUTPUT_SIZE], osc[:MAX_OUTPUT_SIZE]


# speedup vs baseline: 15.6552x; 1.0168x over previous
import functools

import jax
import jax.numpy as jnp
from jax import lax
from jax.experimental import pallas as pl
from jax.experimental.pallas import tpu as pltpu
from jax.experimental.pallas import tpu_sc as plsc

MAX_OUTPUT_SIZE = 100
IOU_THRESHOLD = 0.5
SCORE_THRESHOLD = -1e9
NEG = -1e30

N_BOXES = 20000
NPAD = 20480
NSUB = 16
CHUNK = NPAD // NSUB
TAIL = N_BOXES - (NSUB - 1) * CHUNK
NV = CHUNK // 16
NOUTV = 7
OUTPAD = 112
BIGI = 1 << 24

_mesh = plsc.VectorSubcoreMesh(
    core_axis_name="c", subcore_axis_name="s", num_cores=1, num_subcores=NSUB
)


@functools.partial(
    pl.kernel,
    out_type=(
        jax.ShapeDtypeStruct((OUTPAD,), jnp.int32),
        jax.ShapeDtypeStruct((OUTPAD,), jnp.float32),
    ),
    mesh=_mesh,
    scratch_types=[
        pltpu.VMEM((CHUNK,), jnp.float32),
        pltpu.VMEM((CHUNK + 16,), jnp.float32),
        pltpu.VMEM((CHUNK + 16,), jnp.float32),
        pltpu.VMEM((CHUNK + 16,), jnp.float32),
        pltpu.VMEM((CHUNK + 16,), jnp.float32),
        pltpu.VMEM((CHUNK,), jnp.float32),
        pltpu.VMEM((16,), jnp.float32),
        pltpu.VMEM((NSUB * 8 + 16,), jnp.float32),
        pltpu.VMEM((OUTPAD,), jnp.int32),
        pltpu.VMEM((OUTPAD,), jnp.float32),
        pltpu.VMEM_SHARED((2 * NSUB * 8,), jnp.float32),
    ],
)
def _nms_sc(
    y1_h, x1_h, y2_h, x2_h, sc_h,
    oidx_h, osc_h,
    work, cy1, cx1, cy2, cx2, areas, pub, cand, oidx_v, osc_v, board,
):
    sid = lax.axis_index("s")
    base = sid * CHUNK
    iota = lax.iota(jnp.int32, 16)
    zeros_i = jnp.zeros((16,), jnp.int32)
    zeros_f = jnp.zeros((16,), jnp.float32)
    iobase = base + iota

    @pl.when(sid < NSUB - 1)
    def _():
        pltpu.sync_copy(sc_h.at[pl.ds(base, CHUNK)], work)
        pltpu.sync_copy(y1_h.at[pl.ds(base, CHUNK)], cy1.at[pl.ds(0, CHUNK)])
        pltpu.sync_copy(x1_h.at[pl.ds(base, CHUNK)], cx1.at[pl.ds(0, CHUNK)])
        pltpu.sync_copy(y2_h.at[pl.ds(base, CHUNK)], cy2.at[pl.ds(0, CHUNK)])
        pltpu.sync_copy(x2_h.at[pl.ds(base, CHUNK)], cx2.at[pl.ds(0, CHUNK)])

    @pl.when(sid == NSUB - 1)
    def _():
        pltpu.sync_copy(sc_h.at[pl.ds(base, TAIL)], work.at[pl.ds(0, TAIL)])
        pltpu.sync_copy(y1_h.at[pl.ds(base, TAIL)], cy1.at[pl.ds(0, TAIL)])
        pltpu.sync_copy(x1_h.at[pl.ds(base, TAIL)], cx1.at[pl.ds(0, TAIL)])
        pltpu.sync_copy(y2_h.at[pl.ds(base, TAIL)], cy2.at[pl.ds(0, TAIL)])
        pltpu.sync_copy(x2_h.at[pl.ds(base, TAIL)], cx2.at[pl.ds(0, TAIL)])
        negv = zeros_f + NEG
        for v in range(TAIL // 16, NV):
            work[pl.ds(v * 16, 16)] = negv

    for v in range(NV):
        s = pl.ds(v * 16, 16)
        areas[s] = jnp.maximum(0.0, cy2[s] - cy1[s]) * jnp.maximum(
            0.0, cx2[s] - cx1[s]
        )

    m0 = jnp.full((16,), -jnp.inf, jnp.float32)
    i0 = zeros_i + BIGI
    for v in range(NV):
        wv = work[pl.ds(v * 16, 16)]
        take = wv > m0
        m0 = jnp.where(take, wv, m0)
        i0 = jnp.where(take, iobase + (v * 16), i0)

    pat0 = (iota & 1) * 8
    pat1 = pat0 + 1
    pairk = iota >> 1
    sels = [pairk == k for k in range(8)]
    pubmasks = [iota == j for j in range(6)]
    perms = [iota ^ sh for sh in (8, 4, 2, 1)]

    def step(i, carry):
        m, idxv = carry
        for perm in perms:
            m2 = m[perm]
            i2 = idxv[perm]
            better = (m2 > m) | ((m2 == m) & (i2 < idxv))
            m = jnp.where(better, m2, m)
            idxv = jnp.where(better, i2, idxv)
        li = (idxv - base)[0]
        py1 = cy1[pl.ds(li, 16)][zeros_i]
        px1 = cx1[pl.ds(li, 16)][zeros_i]
        py2 = cy2[pl.ds(li, 16)][zeros_i]
        px2 = cx2[pl.ds(li, 16)][zeros_i]
        lfv = idxv.astype(jnp.float32)
        pubv = jnp.where(
            pubmasks[0], m,
            jnp.where(pubmasks[1], lfv,
            jnp.where(pubmasks[2], py1,
            jnp.where(pubmasks[3], px1,
            jnp.where(pubmasks[4], py2,
            jnp.where(pubmasks[5], px2, 0.0))))),
        )
        pub[...] = pubv
        boff = (i & 1) * (NSUB * 8)
        pltpu.sync_copy(pub.at[pl.ds(0, 8)], board.at[pl.ds(boff + sid * 8, 8)])
        plsc.subcore_barrier()
        pltpu.sync_copy(
            board.at[pl.ds(boff, NSUB * 8)], cand.at[pl.ds(0, NSUB * 8)]
        )

        vals = zeros_f
        idxf = zeros_f
        for k in range(8):
            rk = cand[pl.ds(k * 16, 16)]
            vals = jnp.where(sels[k], rk[pat0], vals)
            idxf = jnp.where(sels[k], rk[pat1], idxf)
        for perm in perms:
            v2 = vals[perm]
            f2 = idxf[perm]
            better = (v2 > vals) | ((v2 == vals) & (f2 < idxf))
            vals = jnp.where(better, v2, vals)
            idxf = jnp.where(better, f2, idxf)
        best_iv = idxf.astype(jnp.int32)
        wrowv = ((best_iv >> 8) * 52429) >> 18
        rowv = cand[pl.ds(wrowv[0] * 8, 16)]
        gy1 = rowv[zeros_i + 2]
        gx1 = rowv[zeros_i + 3]
        gy2 = rowv[zeros_i + 4]
        gx2 = rowv[zeros_i + 5]

        validv = vals > SCORE_THRESHOLD
        iv = zeros_i + i
        lmask_o = iota == (iv & 15)
        st = i & -16
        ov = oidx_v[pl.ds(st, 16)]
        oidx_v[pl.ds(st, 16)] = jnp.where(
            lmask_o, jnp.where(validv, best_iv, -1), ov
        )
        sv = osc_v[pl.ds(st, 16)]
        osc_v[pl.ds(st, 16)] = jnp.where(
            lmask_o, jnp.where(validv, vals, 0.0), sv
        )

        li_all = best_iv - base
        li0 = li_all[0]

        @pl.when((li0 >= 0) & (li0 < CHUNK))
        def _():
            lwa = (li_all & -16)[0]
            lmask = iota == (best_iv & 15)
            wvv = work[pl.ds(lwa, 16)]
            work[pl.ds(lwa, 16)] = jnp.where(lmask, NEG, wvv)

        bav = jnp.maximum(0.0, gy2 - gy1) * jnp.maximum(0.0, gx2 - gx1)
        nm = jnp.full((16,), -jnp.inf, jnp.float32)
        nidx = zeros_i + BIGI
        for v in range(NV):
            s = pl.ds(v * 16, 16)
            ty1 = jnp.maximum(gy1, cy1[s])
            tx1 = jnp.maximum(gx1, cx1[s])
            ty2 = jnp.minimum(gy2, cy2[s])
            tx2 = jnp.minimum(gx2, cx2[s])
            inter = jnp.maximum(0.0, ty2 - ty1) * (tx2 - tx1)
            union = (bav + areas[s]) - inter
            supp = ((inter + inter) - union) * 16777216.0 > union
            wv = work[s]
            nwv = jnp.where(supp, NEG, wv)
            work[s] = nwv
            take = nwv > nm
            nm = jnp.where(take, nwv, nm)
            nidx = jnp.where(take, v, nidx)
        return nm, (nidx << 4) + iobase

    lax.fori_loop(0, MAX_OUTPUT_SIZE, step, (m0, i0))

    @pl.when((sid == 0) & (lax.axis_index("c") == 0))
    def _():
        pltpu.sync_copy(oidx_v, oidx_h)
        pltpu.sync_copy(osc_v, osc_h)


def kernel(boxes, pred_conf):
    n = boxes.shape[0]
    oidx, osc = _nms_sc(
        boxes[:, 0], boxes[:, 1], boxes[:, 2], boxes[:, 3],
        pred_conf.reshape(n),
    )
    return oidx[:MAX_OUTPUT_SIZE], osc[:MAX_OUTPUT_SIZE]
